# baseline (speedup 1.0000x reference)
import functools

import jax
import jax.numpy as jnp
from jax import lax
from jax.experimental import pallas as pl
from jax.experimental.pallas import tpu as pltpu
from jax.experimental.pallas import tpu_sc as plsc

_NC, _NS = 2, 16
_NW = _NC * _NS


def _sc_copy(in_hbm, out_hbm):
    wid = lax.axis_index("s") * _NC + lax.axis_index("c")
    rows = in_hbm.shape[1] // _NW
    base = wid * rows
    for b in range(in_hbm.shape[0]):
        pltpu.sync_copy(
            in_hbm.at[b, pl.ds(base, rows), :],
            out_hbm.at[b, pl.ds(base, rows), :],
        )


def kernel(inputs_embeds, embed_table):
    del embed_table
    mesh = plsc.VectorSubcoreMesh(core_axis_name="c", subcore_axis_name="s")
    k = functools.partial(
        pl.kernel,
        mesh=mesh,
        out_type=jax.ShapeDtypeStruct(inputs_embeds.shape, inputs_embeds.dtype),
    )(_sc_copy)
    return k(inputs_embeds)


# speedup vs baseline: 14.3986x; 14.3986x over previous
import jax
import jax.numpy as jnp
from jax.experimental import pallas as pl
from jax.experimental.pallas import tpu as pltpu

_GRID = 4


def _copy_kernel(in_ref, out_ref):
    out_ref[...] = in_ref[...]


def kernel(inputs_embeds, embed_table):
    del embed_table
    b, s, h = inputs_embeds.shape
    rows = s // _GRID
    return pl.pallas_call(
        _copy_kernel,
        grid=(_GRID,),
        in_specs=[pl.BlockSpec((b, rows, h), lambda i: (0, i, 0))],
        out_specs=pl.BlockSpec((b, rows, h), lambda i: (0, i, 0)),
        out_shape=jax.ShapeDtypeStruct((b, s, h), inputs_embeds.dtype),
    )(inputs_embeds)


# speedup vs baseline: 15.2899x; 1.0619x over previous
import jax
import jax.numpy as jnp
from jax.experimental import pallas as pl
from jax.experimental.pallas import tpu as pltpu

_GRID = 2


def _copy_kernel(in_ref, out_ref):
    out_ref[...] = in_ref[...]


def kernel(inputs_embeds, embed_table):
    del embed_table
    b, s, h = inputs_embeds.shape
    rows = s // _GRID
    return pl.pallas_call(
        _copy_kernel,
        grid=(_GRID,),
        in_specs=[pl.BlockSpec((b, rows, h), lambda i: (0, i, 0))],
        out_specs=pl.BlockSpec((b, rows, h), lambda i: (0, i, 0)),
        out_shape=jax.ShapeDtypeStruct((b, s, h), inputs_embeds.dtype),
    )(inputs_embeds)
